# baseline (speedup 1.0000x reference)
import functools

import jax
import jax.numpy as jnp
from jax import lax
from jax.experimental import pallas as pl
from jax.experimental.pallas import tpu as pltpu
from jax.experimental.pallas import tpu_sc as plsc

N = 10000
D = 128
E = 320000
NC = 2
NS = 16
NW = NC * NS
CHUNK = 128
N_PAD = 79 * 128
PAD_ROWS = N_PAD - N
NCHUNK = 80
E_PAD = NW * NCHUNK * CHUNK
HPT = 640
N_HIST = NS * HPT
RPT = N_PAD // NS

_mesh = plsc.VectorSubcoreMesh(
    core_axis_name="c", subcore_axis_name="s", num_cores=NC, num_subcores=NS
)


@functools.partial(
    pl.kernel,
    out_type=jax.ShapeDtypeStruct((NC, N_HIST), jnp.float32),
    mesh=_mesh,
    scratch_types=[
        pltpu.VMEM_SHARED((N_HIST,), jnp.float32),
        pltpu.VMEM((HPT,), jnp.float32),
        pltpu.VMEM((CHUNK,), jnp.float32),
        pltpu.VMEM((NCHUNK, CHUNK), jnp.int32),
        pltpu.SemaphoreType.DMA,
    ],
)
def _deg_kernel(dst_hbm, out_hbm, hist, zbuf, ones, idx, sem):
    c = lax.axis_index("c")
    s = lax.axis_index("s")
    for i in range(HPT // 16):
        zbuf[pl.ds(i * 16, 16)] = jnp.zeros((16,), jnp.float32)
    for i in range(CHUNK // 16):
        ones[pl.ds(i * 16, 16)] = jnp.ones((16,), jnp.float32)
    pltpu.sync_copy(dst_hbm.at[c, s], idx)
    pltpu.sync_copy(zbuf, hist.at[pl.ds(s * HPT, HPT)])
    plsc.subcore_barrier()

    def body(j, carry):
        pltpu.async_copy(ones, hist.at[idx.at[j]], sem, add=True)
        return carry

    lax.fori_loop(0, NCHUNK, body, 0)
    pltpu.make_async_copy(dst_hbm.at[c, s], idx, sem).wait()
    plsc.subcore_barrier()
    pltpu.sync_copy(hist.at[pl.ds(s * HPT, HPT)], out_hbm.at[c, pl.ds(s * HPT, HPT)])


def _scatter_body(
    g_hbm, src_hbm, dst_hbm, out_hbm, acc, sbuf, dbuf, rows, isems, jsems, gsems, ssems
):
    c = lax.axis_index("c")
    s = lax.axis_index("s")

    def zrow(i, carry):
        for k in range(D // 16):
            rows[0][i, pl.ds(k * 16, 16)] = jnp.zeros((16,), jnp.float32)
        return carry

    lax.fori_loop(0, CHUNK, zrow, 0)
    base = s * RPT
    rem = RPT % CHUNK
    for r in range(RPT // CHUNK):
        pltpu.sync_copy(rows[0], acc.at[pl.ds(base + r * CHUNK, CHUNK)])
    pltpu.sync_copy(rows[0].at[pl.ds(0, rem)], acc.at[pl.ds(base + RPT - rem, rem)])

    for b in range(2):
        pltpu.sync_copy(src_hbm.at[c, s, b], sbuf[b])
        pltpu.sync_copy(dst_hbm.at[c, s, b], dbuf[b])
        pltpu.async_copy(g_hbm.at[sbuf[b]], rows[b], gsems[b])
    plsc.subcore_barrier()

    def body(i, carry):
        for b in range(2):
            j = 2 * i + b
            pltpu.make_async_copy(g_hbm.at[sbuf[b]], rows[b], gsems[b]).wait()
            pltpu.async_copy(rows[b], acc.at[dbuf[b]], ssems[b], add=True)
        for b in range(2):
            j = 2 * i + b
            pltpu.make_async_copy(rows[b], acc.at[dbuf[b]], ssems[b]).wait()
            pltpu.async_copy(src_hbm.at[c, s, j + 2], sbuf[b], isems[b])
            pltpu.async_copy(dst_hbm.at[c, s, j + 2], dbuf[b], jsems[b])
        for b in range(2):
            j = 2 * i + b
            pltpu.make_async_copy(src_hbm.at[c, s, j + 2], sbuf[b], isems[b]).wait()
            pltpu.make_async_copy(dst_hbm.at[c, s, j + 2], dbuf[b], jsems[b]).wait()
            pltpu.async_copy(g_hbm.at[sbuf[b]], rows[b], gsems[b])
        return carry

    lax.fori_loop(0, NCHUNK // 2 - 1, body, 0)
    for b in range(2):
        pltpu.make_async_copy(g_hbm.at[sbuf[b]], rows[b], gsems[b]).wait()
        pltpu.async_copy(rows[b], acc.at[dbuf[b]], ssems[b], add=True)
    for b in range(2):
        pltpu.make_async_copy(rows[b], acc.at[dbuf[b]], ssems[b]).wait()
    plsc.subcore_barrier()

    for r in range(RPT // CHUNK):
        sl = pl.ds(base + r * CHUNK, CHUNK)
        pltpu.sync_copy(acc.at[sl], out_hbm.at[c, sl])
    sl = pl.ds(base + RPT - rem, rem)
    pltpu.sync_copy(acc.at[sl], out_hbm.at[c, sl])


_scatter_kernel = pl.kernel(
    _scatter_body,
    out_type=jax.ShapeDtypeStruct((NC, N_PAD, D), jnp.float32),
    mesh=_mesh,
    scratch_types=[
        pltpu.VMEM_SHARED((N_PAD, D), jnp.float32),
        [pltpu.VMEM((CHUNK,), jnp.int32)] * 2,
        [pltpu.VMEM((CHUNK,), jnp.int32)] * 2,
        [pltpu.VMEM((CHUNK, D), jnp.float32)] * 2,
        [pltpu.SemaphoreType.DMA] * 2,
        [pltpu.SemaphoreType.DMA] * 2,
        [pltpu.SemaphoreType.DMA] * 2,
        [pltpu.SemaphoreType.DMA] * 2,
    ],
)


def _pre_body(deg_ref, x_ref, w_ref, g_ref, dinv_ref):
    dinv = lax.rsqrt(deg_ref[...] + 1.0)
    g_ref[...] = dinv * jnp.dot(
        x_ref[...], w_ref[...], preferred_element_type=jnp.float32
    )
    dinv_ref[...] = dinv


def _mid_body(a0_ref, a1_ref, g_ref, dinv_ref, b_ref, w_ref, out_ref):
    dv = dinv_ref[...]
    h = jnp.maximum(dv * (a0_ref[...] + a1_ref[...] + g_ref[...]) + b_ref[...], 0.0)
    out_ref[...] = dv * jnp.dot(h, w_ref[...], preferred_element_type=jnp.float32)


def _post_body(a0_ref, a1_ref, g_ref, dinv_ref, b_ref, out_ref):
    out_ref[...] = jnp.maximum(
        dinv_ref[...] * (a0_ref[...] + a1_ref[...] + g_ref[...]) + b_ref[...], 0.0
    )


RBLK = 1264
NBLK_TC = N_PAD // RBLK


def _row_spec(w):
    return pl.BlockSpec((RBLK, w), lambda i: (i, 0))


def _full_spec(h, w):
    return pl.BlockSpec((h, w), lambda i: (0, 0))


_f32 = jnp.float32

_pre_call = pl.pallas_call(
    _pre_body,
    grid=(NBLK_TC,),
    in_specs=[_row_spec(1), _row_spec(D), _full_spec(D, D)],
    out_specs=[_row_spec(D), _row_spec(1)],
    out_shape=[
        jax.ShapeDtypeStruct((N_PAD, D), _f32),
        jax.ShapeDtypeStruct((N_PAD, 1), _f32),
    ],
)

_mid_call = pl.pallas_call(
    _mid_body,
    grid=(NBLK_TC,),
    in_specs=[
        _row_spec(D),
        _row_spec(D),
        _row_spec(D),
        _row_spec(1),
        _full_spec(1, D),
        _full_spec(D, D),
    ],
    out_specs=_row_spec(D),
    out_shape=jax.ShapeDtypeStruct((N_PAD, D), _f32),
)

_post_call = pl.pallas_call(
    _post_body,
    grid=(NBLK_TC,),
    in_specs=[_row_spec(D), _row_spec(D), _row_spec(D), _row_spec(1), _full_spec(1, D)],
    out_specs=_row_spec(D),
    out_shape=jax.ShapeDtypeStruct((N_PAD, D), _f32),
)


def kernel(x, edge_index, W1, b1, W2, b2):
    x_pad = jnp.pad(x, ((0, PAD_ROWS), (0, 0)))
    pad_idx = (N + (jnp.arange(E_PAD - E, dtype=jnp.int32) % PAD_ROWS)).astype(
        jnp.int32
    )
    src_t = jnp.concatenate([edge_index[0], pad_idx]).reshape(NC, NS, NCHUNK, CHUNK)
    dst_t = jnp.concatenate([edge_index[1], pad_idx]).reshape(NC, NS, NCHUNK, CHUNK)

    deg_parts = _deg_kernel(dst_t)
    degsum_col = (deg_parts[0, :N_PAD] + deg_parts[1, :N_PAD])[:, None]

    g1, dinv = _pre_call(degsum_col, x_pad, W1)
    s1 = _scatter_kernel(g1, src_t, dst_t)
    g2 = _mid_call(s1[0], s1[1], g1, dinv, b1[None, :], W2)
    s2 = _scatter_kernel(g2, src_t, dst_t)
    out = _post_call(s2[0], s2[1], g2, dinv, b2[None, :])
    return out[:N]


# speedup vs baseline: 1.2911x; 1.2911x over previous
import functools

import jax
import jax.numpy as jnp
from jax import lax
from jax.experimental import pallas as pl
from jax.experimental.pallas import tpu as pltpu
from jax.experimental.pallas import tpu_sc as plsc

N = 10000
D = 128
E = 320000
NC = 2
NS = 16
NW = NC * NS
CHUNK = 128
N_PAD = 79 * 128
PAD_ROWS = N_PAD - N
NCHUNK = 80
E_PAD = NW * NCHUNK * CHUNK
HPT = 640
N_HIST = NS * HPT
RPT = N_PAD // NS

_mesh = plsc.VectorSubcoreMesh(
    core_axis_name="c", subcore_axis_name="s", num_cores=NC, num_subcores=NS
)


@functools.partial(
    pl.kernel,
    out_type=jax.ShapeDtypeStruct((NC, N_HIST), jnp.float32),
    mesh=_mesh,
    scratch_types=[
        pltpu.VMEM_SHARED((N_HIST,), jnp.float32),
        pltpu.VMEM((HPT,), jnp.float32),
        pltpu.VMEM((CHUNK,), jnp.float32),
        pltpu.VMEM((NCHUNK, CHUNK), jnp.int32),
        pltpu.SemaphoreType.DMA,
    ],
)
def _deg_kernel(dst_hbm, out_hbm, hist, zbuf, ones, idx, sem):
    c = lax.axis_index("c")
    s = lax.axis_index("s")
    for i in range(HPT // 16):
        zbuf[pl.ds(i * 16, 16)] = jnp.zeros((16,), jnp.float32)
    for i in range(CHUNK // 16):
        ones[pl.ds(i * 16, 16)] = jnp.ones((16,), jnp.float32)
    pltpu.sync_copy(dst_hbm.at[c, s], idx)
    pltpu.sync_copy(zbuf, hist.at[pl.ds(s * HPT, HPT)])
    plsc.subcore_barrier()

    def body(j, carry):
        pltpu.async_copy(ones, hist.at[idx.at[j]], sem, add=True)
        return carry

    lax.fori_loop(0, NCHUNK, body, 0)
    pltpu.make_async_copy(dst_hbm.at[c, s], idx, sem).wait()
    plsc.subcore_barrier()
    pltpu.sync_copy(hist.at[pl.ds(s * HPT, HPT)], out_hbm.at[c, pl.ds(s * HPT, HPT)])


SCH = 64
NCH_S = 10240 // SCH
EPT = NCH_S * SCH
NBUF = 4


def _unpack(pidx, sidx, didx, b, j):
    for k in range(SCH // 16):
        w = pidx[pl.ds(j * SCH + k * 16, 16)]
        sidx[b][pl.ds(k * 16, 16)] = jnp.bitwise_and(w, 0x3FFF)
        didx[b][pl.ds(k * 16, 16)] = lax.shift_right_logical(w, 14)


def _scatter_body(g_hbm, sd_hbm, out_hbm, acc, pidx, sidx, didx, rows, gsems, ssems):
    c = lax.axis_index("c")
    s = lax.axis_index("s")

    def zrow(i, carry):
        for k in range(D // 16):
            rows[0][i, pl.ds(k * 16, 16)] = jnp.zeros((16,), jnp.float32)
        return carry

    lax.fori_loop(0, SCH, zrow, 0)
    base = s * RPT
    rem = RPT % SCH
    for r in range(RPT // SCH):
        pltpu.sync_copy(rows[0], acc.at[pl.ds(base + r * SCH, SCH)])
    pltpu.sync_copy(rows[0].at[pl.ds(0, rem)], acc.at[pl.ds(base + RPT - rem, rem)])

    pltpu.sync_copy(sd_hbm.at[c, s], pidx)
    for b in range(NBUF):
        _unpack(pidx, sidx, didx, b, b)
        pltpu.async_copy(g_hbm.at[sidx[b]], rows[b], gsems[b])
    plsc.subcore_barrier()

    def body(i, carry):
        j0 = i * NBUF
        for b in range(NBUF):
            pltpu.make_async_copy(g_hbm.at[sidx[b]], rows[b], gsems[b]).wait()
            pltpu.async_copy(rows[b], acc.at[didx[b]], ssems[b], add=True)
        for b in range(NBUF):
            pltpu.make_async_copy(rows[b], acc.at[didx[b]], ssems[b]).wait()
            _unpack(pidx, sidx, didx, b, j0 + NBUF + b)
            pltpu.async_copy(g_hbm.at[sidx[b]], rows[b], gsems[b])
        return carry

    lax.fori_loop(0, NCH_S // NBUF - 1, body, 0)
    for b in range(NBUF):
        pltpu.make_async_copy(g_hbm.at[sidx[b]], rows[b], gsems[b]).wait()
        pltpu.async_copy(rows[b], acc.at[didx[b]], ssems[b], add=True)
    for b in range(NBUF):
        pltpu.make_async_copy(rows[b], acc.at[didx[b]], ssems[b]).wait()
    plsc.subcore_barrier()

    for r in range(RPT // SCH):
        sl = pl.ds(base + r * SCH, SCH)
        pltpu.sync_copy(acc.at[sl], out_hbm.at[c, sl])
    sl = pl.ds(base + RPT - rem, rem)
    pltpu.sync_copy(acc.at[sl], out_hbm.at[c, sl])


_scatter_kernel = pl.kernel(
    _scatter_body,
    out_type=jax.ShapeDtypeStruct((NC, N_PAD, D), jnp.float32),
    mesh=_mesh,
    scratch_types=[
        pltpu.VMEM_SHARED((N_PAD, D), jnp.float32),
        pltpu.VMEM((EPT,), jnp.int32),
        [pltpu.VMEM((SCH,), jnp.int32)] * NBUF,
        [pltpu.VMEM((SCH,), jnp.int32)] * NBUF,
        [pltpu.VMEM((SCH, D), jnp.float32)] * NBUF,
        [pltpu.SemaphoreType.DMA] * NBUF,
        [pltpu.SemaphoreType.DMA] * NBUF,
    ],
)


def _pre_body(deg_ref, x_ref, w_ref, g_ref, dinv_ref):
    dinv = lax.rsqrt(deg_ref[...] + 1.0)
    g_ref[...] = dinv * jnp.dot(
        x_ref[...], w_ref[...], preferred_element_type=jnp.float32
    )
    dinv_ref[...] = dinv


def _mid_body(a0_ref, a1_ref, g_ref, dinv_ref, b_ref, w_ref, out_ref):
    dv = dinv_ref[...]
    h = jnp.maximum(dv * (a0_ref[...] + a1_ref[...] + g_ref[...]) + b_ref[...], 0.0)
    out_ref[...] = dv * jnp.dot(h, w_ref[...], preferred_element_type=jnp.float32)


def _post_body(a0_ref, a1_ref, g_ref, dinv_ref, b_ref, out_ref):
    out_ref[...] = jnp.maximum(
        dinv_ref[...] * (a0_ref[...] + a1_ref[...] + g_ref[...]) + b_ref[...], 0.0
    )


RBLK = 1264
NBLK_TC = N_PAD // RBLK


def _row_spec(w):
    return pl.BlockSpec((RBLK, w), lambda i: (i, 0))


def _full_spec(h, w):
    return pl.BlockSpec((h, w), lambda i: (0, 0))


_f32 = jnp.float32

_pre_call = pl.pallas_call(
    _pre_body,
    grid=(NBLK_TC,),
    in_specs=[_row_spec(1), _row_spec(D), _full_spec(D, D)],
    out_specs=[_row_spec(D), _row_spec(1)],
    out_shape=[
        jax.ShapeDtypeStruct((N_PAD, D), _f32),
        jax.ShapeDtypeStruct((N_PAD, 1), _f32),
    ],
)

_mid_call = pl.pallas_call(
    _mid_body,
    grid=(NBLK_TC,),
    in_specs=[
        _row_spec(D),
        _row_spec(D),
        _row_spec(D),
        _row_spec(1),
        _full_spec(1, D),
        _full_spec(D, D),
    ],
    out_specs=_row_spec(D),
    out_shape=jax.ShapeDtypeStruct((N_PAD, D), _f32),
)

_post_call = pl.pallas_call(
    _post_body,
    grid=(NBLK_TC,),
    in_specs=[_row_spec(D), _row_spec(D), _row_spec(D), _row_spec(1), _full_spec(1, D)],
    out_specs=_row_spec(D),
    out_shape=jax.ShapeDtypeStruct((N_PAD, D), _f32),
)


def kernel(x, edge_index, W1, b1, W2, b2):
    x_pad = jnp.pad(x, ((0, PAD_ROWS), (0, 0)))
    pad_idx = (N + (jnp.arange(E_PAD - E, dtype=jnp.int32) % PAD_ROWS)).astype(
        jnp.int32
    )
    src = jnp.concatenate([edge_index[0], pad_idx])
    dst = jnp.concatenate([edge_index[1], pad_idx])
    dst_t = dst.reshape(NC, NS, NCHUNK, CHUNK)
    sd = (jnp.left_shift(dst, 14) | src).reshape(NC, NS, EPT)

    deg_parts = _deg_kernel(dst_t)
    degsum_col = (deg_parts[0, :N_PAD] + deg_parts[1, :N_PAD])[:, None]

    g1, dinv = _pre_call(degsum_col, x_pad, W1)
    s1 = _scatter_kernel(g1, sd)
    g2 = _mid_call(s1[0], s1[1], g1, dinv, b1[None, :], W2)
    s2 = _scatter_kernel(g2, sd)
    out = _post_call(s2[0], s2[1], g2, dinv, b2[None, :])
    return out[:N]


# speedup vs baseline: 1.2918x; 1.0006x over previous
import functools

import jax
import jax.numpy as jnp
from jax import lax
from jax.experimental import pallas as pl
from jax.experimental.pallas import tpu as pltpu
from jax.experimental.pallas import tpu_sc as plsc

N = 10000
D = 128
E = 320000
NC = 2
NS = 16
NW = NC * NS
CHUNK = 128
N_PAD = 79 * 128
PAD_ROWS = N_PAD - N
NCHUNK = 80
E_PAD = NW * NCHUNK * CHUNK
HPT = 640
N_HIST = NS * HPT
RPT = N_PAD // NS

_mesh = plsc.VectorSubcoreMesh(
    core_axis_name="c", subcore_axis_name="s", num_cores=NC, num_subcores=NS
)


@functools.partial(
    pl.kernel,
    out_type=jax.ShapeDtypeStruct((NC, N_HIST), jnp.float32),
    mesh=_mesh,
    scratch_types=[
        pltpu.VMEM_SHARED((N_HIST,), jnp.float32),
        pltpu.VMEM((HPT,), jnp.float32),
        pltpu.VMEM((CHUNK,), jnp.float32),
        pltpu.VMEM((NCHUNK, CHUNK), jnp.int32),
        pltpu.SemaphoreType.DMA,
    ],
)
def _deg_kernel(dst_hbm, out_hbm, hist, zbuf, ones, idx, sem):
    c = lax.axis_index("c")
    s = lax.axis_index("s")
    for i in range(HPT // 16):
        zbuf[pl.ds(i * 16, 16)] = jnp.zeros((16,), jnp.float32)
    for i in range(CHUNK // 16):
        ones[pl.ds(i * 16, 16)] = jnp.ones((16,), jnp.float32)
    pltpu.sync_copy(dst_hbm.at[c, s], idx)
    pltpu.sync_copy(zbuf, hist.at[pl.ds(s * HPT, HPT)])
    plsc.subcore_barrier()

    def body(j, carry):
        pltpu.async_copy(ones, hist.at[idx.at[j]], sem, add=True)
        return carry

    lax.fori_loop(0, NCHUNK, body, 0)
    pltpu.make_async_copy(dst_hbm.at[c, s], idx, sem).wait()
    plsc.subcore_barrier()
    pltpu.sync_copy(hist.at[pl.ds(s * HPT, HPT)], out_hbm.at[c, pl.ds(s * HPT, HPT)])


SCH = 64
NCH_S = 10240 // SCH
EPT = NCH_S * SCH
NBUF = 4


def _unpack(pidx, sidx, didx, b, j):
    for k in range(SCH // 16):
        w = pidx[pl.ds(j * SCH + k * 16, 16)]
        sidx[b][pl.ds(k * 16, 16)] = jnp.bitwise_and(w, 0x3FFF)
        didx[b][pl.ds(k * 16, 16)] = lax.shift_right_logical(w, 14)


def _scatter_body(g_hbm, sd_hbm, out_hbm, acc, pidx, sidx, didx, rows, gsems, ssems):
    c = lax.axis_index("c")
    s = lax.axis_index("s")

    def zrow(i, carry):
        for k in range(D // 16):
            rows[0][i, pl.ds(k * 16, 16)] = jnp.zeros((16,), jnp.float32)
        return carry

    lax.fori_loop(0, SCH, zrow, 0)
    base = s * RPT
    rem = RPT % SCH
    for r in range(RPT // SCH):
        pltpu.sync_copy(rows[0], acc.at[pl.ds(base + r * SCH, SCH)])
    pltpu.sync_copy(rows[0].at[pl.ds(0, rem)], acc.at[pl.ds(base + RPT - rem, rem)])

    pltpu.sync_copy(sd_hbm.at[c, s], pidx)
    for b in range(NBUF):
        _unpack(pidx, sidx, didx, b, b)
        pltpu.async_copy(g_hbm.at[sidx[b]], rows[b], gsems[b])
    plsc.subcore_barrier()

    def body(i, carry):
        j0 = i * NBUF
        for b in range(NBUF):
            pltpu.make_async_copy(g_hbm.at[sidx[b]], rows[b], gsems[b]).wait()
            pltpu.async_copy(rows[b], acc.at[didx[b]], ssems[b], add=True)
        for b in range(NBUF):
            pltpu.make_async_copy(rows[b], acc.at[didx[b]], ssems[b]).wait()
            _unpack(pidx, sidx, didx, b, j0 + NBUF + b)
            pltpu.async_copy(g_hbm.at[sidx[b]], rows[b], gsems[b])
        return carry

    lax.fori_loop(0, NCH_S // NBUF - 1, body, 0)
    for b in range(NBUF):
        pltpu.make_async_copy(g_hbm.at[sidx[b]], rows[b], gsems[b]).wait()
        pltpu.async_copy(rows[b], acc.at[didx[b]], ssems[b], add=True)
    for b in range(NBUF):
        pltpu.make_async_copy(rows[b], acc.at[didx[b]], ssems[b]).wait()
    plsc.subcore_barrier()

    for r in range(RPT // SCH):
        sl = pl.ds(base + r * SCH, SCH)
        pltpu.sync_copy(acc.at[sl], out_hbm.at[c, sl])
    sl = pl.ds(base + RPT - rem, rem)
    pltpu.sync_copy(acc.at[sl], out_hbm.at[c, sl])


_scatter_kernel = pl.kernel(
    _scatter_body,
    out_type=jax.ShapeDtypeStruct((NC, N_PAD, D), jnp.float32),
    mesh=_mesh,
    scratch_types=[
        pltpu.VMEM_SHARED((N_PAD, D), jnp.float32),
        pltpu.VMEM((EPT,), jnp.int32),
        [pltpu.VMEM((SCH,), jnp.int32)] * NBUF,
        [pltpu.VMEM((SCH,), jnp.int32)] * NBUF,
        [pltpu.VMEM((SCH, D), jnp.float32)] * NBUF,
        [pltpu.SemaphoreType.DMA] * NBUF,
        [pltpu.SemaphoreType.DMA] * NBUF,
    ],
)


def _pre_body(deg_ref, x_ref, w_ref, g_ref, dinv_ref):
    dinv = lax.rsqrt(deg_ref[...] + 1.0)
    g_ref[...] = dinv * jnp.dot(
        x_ref[...], w_ref[...], preferred_element_type=jnp.float32
    )
    dinv_ref[...] = dinv


def _mid_body(a0_ref, a1_ref, g_ref, dinv_ref, b_ref, w_ref, out_ref):
    dv = dinv_ref[...]
    h = jnp.maximum(dv * (a0_ref[...] + a1_ref[...] + g_ref[...]) + b_ref[...], 0.0)
    out_ref[...] = dv * jnp.dot(h, w_ref[...], preferred_element_type=jnp.float32)


def _post_body(a0_ref, a1_ref, g_ref, dinv_ref, b_ref, out_ref):
    out_ref[...] = jnp.maximum(
        dinv_ref[...] * (a0_ref[...] + a1_ref[...] + g_ref[...]) + b_ref[...], 0.0
    )


RBLK = 1264
NBLK_TC = N_PAD // RBLK


def _row_spec(w):
    return pl.BlockSpec((RBLK, w), lambda i: (i, 0))


def _full_spec(h, w):
    return pl.BlockSpec((h, w), lambda i: (0, 0))


_f32 = jnp.float32

_pre_call = pl.pallas_call(
    _pre_body,
    grid=(NBLK_TC,),
    in_specs=[_row_spec(1), _row_spec(D), _full_spec(D, D)],
    out_specs=[_row_spec(D), _row_spec(1)],
    out_shape=[
        jax.ShapeDtypeStruct((N_PAD, D), _f32),
        jax.ShapeDtypeStruct((N_PAD, 1), _f32),
    ],
)

_mid_call = pl.pallas_call(
    _mid_body,
    grid=(NBLK_TC,),
    in_specs=[
        _row_spec(D),
        _row_spec(D),
        _row_spec(D),
        _row_spec(1),
        _full_spec(1, D),
        _full_spec(D, D),
    ],
    out_specs=_row_spec(D),
    out_shape=jax.ShapeDtypeStruct((N_PAD, D), _f32),
)

_post_call = pl.pallas_call(
    _post_body,
    grid=(NBLK_TC,),
    in_specs=[_row_spec(D), _row_spec(D), _row_spec(D), _row_spec(1), _full_spec(1, D)],
    out_specs=_row_spec(D),
    out_shape=jax.ShapeDtypeStruct((N_PAD, D), _f32),
)


def kernel(x, edge_index, W1, b1, W2, b2):
    pad_idx = (N + (jnp.arange(E_PAD - E, dtype=jnp.int32) % PAD_ROWS)).astype(
        jnp.int32
    )
    src = jnp.concatenate([edge_index[0], pad_idx])
    dst = jnp.concatenate([edge_index[1], pad_idx])
    dst_t = dst.reshape(NC, NS, NCHUNK, CHUNK)
    sd = (jnp.left_shift(dst, 14) | src).reshape(NC, NS, EPT)

    deg_parts = _deg_kernel(dst_t)
    degsum_col = (deg_parts[0, :N_PAD] + deg_parts[1, :N_PAD])[:, None]

    g1, dinv = _pre_call(degsum_col, x, W1)
    s1 = _scatter_kernel(g1, sd)
    g2 = _mid_call(s1[0], s1[1], g1, dinv, b1[None, :], W2)
    s2 = _scatter_kernel(g2, sd)
    out = _post_call(s2[0], s2[1], g2, dinv, b2[None, :])
    return out[:N]
